# baseline (speedup 1.0000x reference)
import functools

import numpy as np

import jax
import jax.numpy as jnp
from jax import lax
from jax.experimental import pallas as pl
from jax.experimental.pallas import tpu as pltpu
from jax.experimental.pallas import tpu_sc as plsc

_MAX_ROWS = 30
_MAX_COLS = 30
_NUM_COLORS = 10
_HIDDEN = 512
_HALF = _HIDDEN // 2
_EPS = 1e-5
_NVALID = _MAX_ROWS * _MAX_COLS * _NUM_COLORS
_NINV = _MAX_ROWS * _MAX_COLS
_ROWS_PER_R = _MAX_COLS * _NUM_COLORS
_HI_MASK = np.int32(-65536)


def _ln(x, g, b):
    mu = jnp.mean(x, axis=-1, keepdims=True)
    xc = x - mu
    var = jnp.mean(xc * xc, axis=-1, keepdims=True)
    return xc * lax.rsqrt(var + _EPS) * g + b


def _pack(x):
    ar = x[..., :_HALF].astype(jnp.bfloat16).astype(jnp.float32)
    br = x[..., _HALF:].astype(jnp.bfloat16).astype(jnp.float32)
    ai = lax.bitcast_convert_type(ar, jnp.int32)
    bi = lax.bitcast_convert_type(br, jnp.int32)
    return lax.shift_right_logical(ai, 16) | (bi & _HI_MASK)


def _build_tables_body(tbl_ref, row_ref, cc_ref, colb_ref, gamma_ref, beta_ref,
                       outv_ref, outi_ref):
    g = gamma_ref[...]
    b = beta_ref[...]
    outv_ref[...] = _pack(_ln(tbl_ref[...] + row_ref[...] + cc_ref[...], g, b))
    outi_ref[...] = _pack(_ln(row_ref[...] + colb_ref[...], g, b))


def _build_fused_tables(tbl, row_table, cc, colb, gamma2, beta2):
    tbl3 = tbl.reshape(_MAX_ROWS, _ROWS_PER_R, _HIDDEN)
    row3 = row_table.reshape(_MAX_ROWS, 1, _HIDDEN)
    outv, outi = pl.pallas_call(
        _build_tables_body,
        grid=(_MAX_ROWS,),
        in_specs=[
            pl.BlockSpec((1, _ROWS_PER_R, _HIDDEN), lambda i: (i, 0, 0)),
            pl.BlockSpec((1, 1, _HIDDEN), lambda i: (i, 0, 0)),
            pl.BlockSpec((_ROWS_PER_R, _HIDDEN), lambda i: (0, 0)),
            pl.BlockSpec((_MAX_COLS, _HIDDEN), lambda i: (0, 0)),
            pl.BlockSpec((1, _HIDDEN), lambda i: (0, 0)),
            pl.BlockSpec((1, _HIDDEN), lambda i: (0, 0)),
        ],
        out_specs=[
            pl.BlockSpec((1, _ROWS_PER_R, _HALF), lambda i: (i, 0, 0)),
            pl.BlockSpec((1, _MAX_COLS, _HALF), lambda i: (i, 0, 0)),
        ],
        out_shape=[
            jax.ShapeDtypeStruct((_MAX_ROWS, _ROWS_PER_R, _HALF), jnp.int32),
            jax.ShapeDtypeStruct((_MAX_ROWS, _MAX_COLS, _HALF), jnp.int32),
        ],
    )(tbl3, row3, cc, colb, gamma2, beta2)
    return outv.reshape(_NVALID, _HALF), outi.reshape(_NINV, _HALF)


def _sc_gather(fused, idxp, bsz, hwp):
    info = plsc.get_sparse_core_info()
    nc, ns = info.num_cores, info.num_subcores
    nw = nc * ns
    d = fused.shape[1]
    n = bsz * hwp
    bpw = n // nw
    ch = 128
    nb = 2
    steps = bpw // ch
    groups = steps // nb
    mesh = plsc.VectorSubcoreMesh(core_axis_name="c", subcore_axis_name="s")

    @functools.partial(
        pl.kernel,
        mesh=mesh,
        out_type=jax.ShapeDtypeStruct((n, d), jnp.int32),
        scratch_types=(
            [pltpu.VMEM((bpw,), jnp.int32)]
            + [pltpu.VMEM((ch, d), jnp.int32) for _ in range(nb)]
            + [pltpu.SemaphoreType.DMA for _ in range(2 * nb)]
        ),
    )
    def k(fused_hbm, idx_hbm, out_hbm, idx_v, *rest):
        bufs = rest[:nb]
        gsems = rest[nb:2 * nb]
        osems = rest[2 * nb:]
        wid = lax.axis_index("s") * nc + lax.axis_index("c")
        base = wid * bpw
        pltpu.sync_copy(idx_hbm.at[pl.ds(base, bpw)], idx_v)

        def gather_desc(g, b):
            return pltpu.make_async_copy(
                fused_hbm.at[idx_v.at[pl.ds(g * ch, ch)]], bufs[b], gsems[b])

        def out_desc(g, b):
            return pltpu.make_async_copy(
                bufs[b], out_hbm.at[pl.ds(base + g * ch, ch)], osems[b])

        for b in range(nb):
            gather_desc(b, b).start()

        def body(t, carry):
            g0 = t * nb
            for b in range(nb):
                gather_desc(g0 + b, b).wait()
                out_desc(g0 + b, b).start()
            for b in range(nb):
                out_desc(g0 + b, b).wait()

                @pl.when(g0 + nb + b < steps)
                def _():
                    gather_desc(g0 + nb + b, b).start()
            return carry

        lax.fori_loop(0, groups, body, 0)

    return k(fused, idxp)


def _expand_body(in_ref, out_ref):
    hw = out_ref.shape[1]
    x = in_ref[:, :hw, :]
    a = lax.bitcast_convert_type(x << 16, jnp.float32)
    b = lax.bitcast_convert_type(x & _HI_MASK, jnp.float32)
    out_ref[...] = jnp.concatenate([a, b], axis=-1)


def _expand(rows_p, bsz, hw, hwp, gb=4):
    return pl.pallas_call(
        _expand_body,
        grid=(bsz // gb,),
        in_specs=[pl.BlockSpec((gb, hwp, _HALF), lambda i: (i, 0, 0))],
        out_specs=pl.BlockSpec((gb, hw, _HIDDEN), lambda i: (i, 0, 0)),
        out_shape=jax.ShapeDtypeStruct((bsz, hw, _HIDDEN), jnp.float32),
    )(rows_p)


def kernel(color_grid, valid_mask, coord_color_table, row_table, col_table,
           W_color, b_color, W_valid, b_valid, ln_gamma, ln_beta):
    bsz, h, w = color_grid.shape
    f32 = jnp.float32

    bias = (b_color + b_valid).astype(f32)
    wc_rows = W_color.T.astype(f32)
    wv_row = W_valid[:, 0].astype(f32)
    cc = (col_table[:, None, :] + wc_rows[None, :, :]).reshape(_ROWS_PER_R, _HIDDEN)
    cc = cc + (wv_row + bias)[None, :]
    colb = col_table + bias[None, :]
    gamma2 = ln_gamma.reshape(1, _HIDDEN).astype(f32)
    beta2 = ln_beta.reshape(1, _HIDDEN).astype(f32)
    tbl = coord_color_table[:_NVALID].astype(f32)

    fused_v, fused_i = _build_fused_tables(tbl, row_table.astype(f32), cc, colb,
                                           gamma2, beta2)
    fused = jnp.concatenate([fused_v, fused_i], axis=0)

    r_ids = jnp.arange(h, dtype=jnp.int32)
    c_ids = jnp.arange(w, dtype=jnp.int32)
    token = (r_ids[None, :, None] * (_MAX_COLS * _NUM_COLORS)
             + c_ids[None, None, :] * _NUM_COLORS
             + color_grid.astype(jnp.int32))
    inv = _NVALID + r_ids[None, :, None] * _MAX_COLS + c_ids[None, None, :]
    idx = jnp.where(valid_mask, token, inv).astype(jnp.int32).reshape(bsz, h * w)

    hw = h * w
    hwp = ((hw + 7) // 8) * 8
    idxp = jnp.concatenate(
        [idx, jnp.zeros((bsz, hwp - hw), jnp.int32)], axis=1).reshape(-1)

    rows = _sc_gather(fused, idxp, bsz, hwp)
    rows_p = rows.reshape(bsz, hwp, _HALF)
    return _expand(rows_p, bsz, hw, hwp)


# speedup vs baseline: 1.0294x; 1.0294x over previous
import functools

import jax
import jax.numpy as jnp
from jax import lax
from jax.experimental import pallas as pl
from jax.experimental.pallas import tpu as pltpu
from jax.experimental.pallas import tpu_sc as plsc

_MAX_ROWS = 30
_MAX_COLS = 30
_NUM_COLORS = 10
_HIDDEN = 512
_EPS = 1e-5
_NVALID = _MAX_ROWS * _MAX_COLS * _NUM_COLORS
_NINV = _MAX_ROWS * _MAX_COLS
_ROWS_PER_R = _MAX_COLS * _NUM_COLORS


def _ln(x, g, b):
    mu = jnp.mean(x, axis=-1, keepdims=True)
    xc = x - mu
    var = jnp.mean(xc * xc, axis=-1, keepdims=True)
    return xc * lax.rsqrt(var + _EPS) * g + b


def _build_tables_body(tbl_ref, row_ref, cc_ref, colb_ref, gamma_ref, beta_ref,
                       outv_ref, outi_ref):
    g = gamma_ref[...]
    b = beta_ref[...]
    outv_ref[...] = _ln(tbl_ref[...] + row_ref[...] + cc_ref[...], g, b)
    outi_ref[...] = _ln(row_ref[...] + colb_ref[...], g, b)


def _build_fused_tables(tbl, row_table, cc, colb, gamma2, beta2):
    tbl3 = tbl.reshape(_MAX_ROWS, _ROWS_PER_R, _HIDDEN)
    row3 = row_table.reshape(_MAX_ROWS, 1, _HIDDEN)
    outv, outi = pl.pallas_call(
        _build_tables_body,
        grid=(_MAX_ROWS,),
        in_specs=[
            pl.BlockSpec((1, _ROWS_PER_R, _HIDDEN), lambda i: (i, 0, 0)),
            pl.BlockSpec((1, 1, _HIDDEN), lambda i: (i, 0, 0)),
            pl.BlockSpec((_ROWS_PER_R, _HIDDEN), lambda i: (0, 0)),
            pl.BlockSpec((_MAX_COLS, _HIDDEN), lambda i: (0, 0)),
            pl.BlockSpec((1, _HIDDEN), lambda i: (0, 0)),
            pl.BlockSpec((1, _HIDDEN), lambda i: (0, 0)),
        ],
        out_specs=[
            pl.BlockSpec((1, _ROWS_PER_R, _HIDDEN), lambda i: (i, 0, 0)),
            pl.BlockSpec((1, _MAX_COLS, _HIDDEN), lambda i: (i, 0, 0)),
        ],
        out_shape=[
            jax.ShapeDtypeStruct((_MAX_ROWS, _ROWS_PER_R, _HIDDEN), jnp.float32),
            jax.ShapeDtypeStruct((_MAX_ROWS, _MAX_COLS, _HIDDEN), jnp.float32),
        ],
    )(tbl3, row3, cc, colb, gamma2, beta2)
    return outv.reshape(_NVALID, _HIDDEN), outi.reshape(_NINV, _HIDDEN)


def _sc_gather(fused, idx_t):
    info = plsc.get_sparse_core_info()
    nc, ns = info.num_cores, info.num_subcores
    nw = nc * ns
    d = fused.shape[1]
    n = idx_t.shape[0]
    bpw = n // nw
    ch = 96
    nb = 2
    steps = bpw // ch
    groups = steps // nb
    mesh = plsc.VectorSubcoreMesh(core_axis_name="c", subcore_axis_name="s")

    @functools.partial(
        pl.kernel,
        mesh=mesh,
        out_type=jax.ShapeDtypeStruct((n, d), jnp.float32),
        scratch_types=(
            [pltpu.VMEM((bpw,), jnp.int32)]
            + [pltpu.VMEM((ch, d), jnp.float32) for _ in range(nb)]
            + [pltpu.SemaphoreType.DMA for _ in range(2 * nb)]
        ),
    )
    def k(fused_hbm, idx_hbm, out_hbm, idx_v, *rest):
        bufs = rest[:nb]
        gsems = rest[nb:2 * nb]
        osems = rest[2 * nb:]
        wid = lax.axis_index("s") * nc + lax.axis_index("c")
        base = wid * bpw
        pltpu.sync_copy(idx_hbm.at[pl.ds(base, bpw)], idx_v)

        def gather_desc(g, b):
            return pltpu.make_async_copy(
                fused_hbm.at[idx_v.at[pl.ds(g * ch, ch)]], bufs[b], gsems[b])

        def out_desc(g, b):
            return pltpu.make_async_copy(
                bufs[b], out_hbm.at[pl.ds(base + g * ch, ch)], osems[b])

        for b in range(nb):
            gather_desc(b, b).start()

        def body(t, carry):
            g0 = t * nb
            for b in range(nb):
                gather_desc(g0 + b, b).wait()
                out_desc(g0 + b, b).start()
            for b in range(nb):
                out_desc(g0 + b, b).wait()

                @pl.when(g0 + nb + b < steps)
                def _():
                    gather_desc(g0 + nb + b, b).start()
            return carry

        lax.fori_loop(0, groups, body, 0)

    return k(fused, idx_t)


def kernel(color_grid, valid_mask, coord_color_table, row_table, col_table,
           W_color, b_color, W_valid, b_valid, ln_gamma, ln_beta):
    bsz, h, w = color_grid.shape
    f32 = jnp.float32

    bias = (b_color + b_valid).astype(f32)
    wc_rows = W_color.T.astype(f32)
    wv_row = W_valid[:, 0].astype(f32)
    cc = (col_table[:, None, :] + wc_rows[None, :, :]).reshape(_ROWS_PER_R, _HIDDEN)
    cc = cc + (wv_row + bias)[None, :]
    colb = col_table + bias[None, :]
    gamma2 = ln_gamma.reshape(1, _HIDDEN).astype(f32)
    beta2 = ln_beta.reshape(1, _HIDDEN).astype(f32)
    tbl = coord_color_table[:_NVALID].astype(f32)

    fused_v, fused_i = _build_fused_tables(tbl, row_table.astype(f32), cc, colb,
                                           gamma2, beta2)
    fused = jnp.concatenate([fused_v, fused_i], axis=0)
    fused = fused.reshape(990, 10, _HIDDEN).transpose(1, 0, 2).reshape(
        _NVALID + _NINV, _HIDDEN)

    r_ids = jnp.arange(h, dtype=jnp.int32)
    c_ids = jnp.arange(w, dtype=jnp.int32)
    cg_t = color_grid.astype(jnp.int32).transpose(1, 2, 0)
    vm_t = valid_mask.transpose(1, 2, 0)
    token = (r_ids[:, None, None] * (_MAX_COLS * _NUM_COLORS)
             + c_ids[None, :, None] * _NUM_COLORS
             + cg_t)
    inv = _NVALID + r_ids[:, None, None] * _MAX_COLS + c_ids[None, :, None]
    idx_t = jnp.where(vm_t, token, inv).astype(jnp.int32).reshape(-1)
    idx_t = (idx_t % 10) * 990 + idx_t // 10

    rows = _sc_gather(fused, idx_t)
    return rows.reshape(h * w, bsz, _HIDDEN).swapaxes(0, 1)


# speedup vs baseline: 2.2451x; 2.1810x over previous
import functools

import jax
import jax.numpy as jnp
from jax import lax
from jax.experimental import pallas as pl
from jax.experimental.pallas import tpu as pltpu
from jax.experimental.pallas import tpu_sc as plsc

_MAX_ROWS = 30
_MAX_COLS = 30
_NUM_COLORS = 10
_HIDDEN = 512
_EPS = 1e-5
_NVALID = _MAX_ROWS * _MAX_COLS * _NUM_COLORS
_NINV = _MAX_ROWS * _MAX_COLS
_ROWS_PER_R = _MAX_COLS * _NUM_COLORS


def _ln(x, g, b):
    mu = jnp.mean(x, axis=-1, keepdims=True)
    xc = x - mu
    var = jnp.mean(xc * xc, axis=-1, keepdims=True)
    return xc * lax.rsqrt(var + _EPS) * g + b


def _build_tables_body(tbl_ref, row_ref, cc_ref, colb_ref, gamma_ref, beta_ref,
                       outv_ref, outi_ref):
    g = gamma_ref[...]
    b = beta_ref[...]
    outv_ref[...] = _ln(tbl_ref[...] + row_ref[...] + cc_ref[...], g, b)
    outi_ref[...] = _ln(row_ref[...] + colb_ref[...], g, b)


def _build_fused_tables(tbl, row_table, cc, colb, gamma2, beta2):
    tbl3 = tbl.reshape(_MAX_ROWS, _ROWS_PER_R, _HIDDEN)
    row3 = row_table.reshape(_MAX_ROWS, 1, _HIDDEN)
    outv, outi = pl.pallas_call(
        _build_tables_body,
        grid=(_MAX_ROWS,),
        in_specs=[
            pl.BlockSpec((1, _ROWS_PER_R, _HIDDEN), lambda i: (i, 0, 0)),
            pl.BlockSpec((1, 1, _HIDDEN), lambda i: (i, 0, 0)),
            pl.BlockSpec((_ROWS_PER_R, _HIDDEN), lambda i: (0, 0)),
            pl.BlockSpec((_MAX_COLS, _HIDDEN), lambda i: (0, 0)),
            pl.BlockSpec((1, _HIDDEN), lambda i: (0, 0)),
            pl.BlockSpec((1, _HIDDEN), lambda i: (0, 0)),
        ],
        out_specs=[
            pl.BlockSpec((1, _ROWS_PER_R, _HIDDEN), lambda i: (i, 0, 0)),
            pl.BlockSpec((1, _MAX_COLS, _HIDDEN), lambda i: (i, 0, 0)),
        ],
        out_shape=[
            jax.ShapeDtypeStruct((_MAX_ROWS, _ROWS_PER_R, _HIDDEN), jnp.float32),
            jax.ShapeDtypeStruct((_MAX_ROWS, _MAX_COLS, _HIDDEN), jnp.float32),
        ],
    )(tbl3, row3, cc, colb, gamma2, beta2)
    return outv.reshape(_NVALID, _HIDDEN), outi.reshape(_NINV, _HIDDEN)


def _sc_gather(fused, idx_t):
    info = plsc.get_sparse_core_info()
    nc, ns = info.num_cores, info.num_subcores
    nw = nc * ns
    d = fused.shape[1]
    n = idx_t.shape[0]
    bpw = n // nw
    ch = 64
    nb = 2
    steps = bpw // ch
    groups = steps // nb
    mesh = plsc.VectorSubcoreMesh(core_axis_name="c", subcore_axis_name="s")

    idx3 = idx_t.reshape(nw, ch, steps).transpose(0, 2, 1)

    @functools.partial(
        pl.kernel,
        mesh=mesh,
        out_type=jax.ShapeDtypeStruct((n, d), jnp.float32),
        scratch_types=(
            [pltpu.VMEM((steps, ch), jnp.int32)]
            + [pltpu.VMEM((ch,), jnp.int32) for _ in range(nb)]
            + [pltpu.VMEM((ch, d), jnp.float32) for _ in range(nb)]
            + [pltpu.SemaphoreType.DMA for _ in range(2 * nb)]
        ),
    )
    def k(fused_hbm, idx_hbm, out_hbm, idx_v, *rest):
        poss = rest[:nb]
        bufs = rest[nb:2 * nb]
        gsems = rest[2 * nb:3 * nb]
        osems = rest[3 * nb:]
        wid = lax.axis_index("s") * nc + lax.axis_index("c")
        base = wid * bpw
        pltpu.sync_copy(idx_hbm.at[wid], idx_v)

        def gather_desc(g, b):
            return pltpu.make_async_copy(
                fused_hbm.at[idx_v.at[g]], bufs[b], gsems[b])

        def out_desc(b):
            return pltpu.make_async_copy(
                bufs[b], out_hbm.at[poss[b]], osems[b])

        def fill_pos(g, b):
            for kk in range(ch // 16):
                poss[b][pl.ds(kk * 16, 16)] = (
                    lax.iota(jnp.int32, 16) * steps + (base + g + kk * 16 * steps))

        for b in range(nb):
            gather_desc(b, b).start()

        def body(t, carry):
            g0 = t * nb
            for b in range(nb):
                gather_desc(g0 + b, b).wait()
                fill_pos(g0 + b, b)
                out_desc(b).start()
            for b in range(nb):
                out_desc(b).wait()

                @pl.when(g0 + nb + b < steps)
                def _():
                    gather_desc(g0 + nb + b, b).start()
            return carry

        lax.fori_loop(0, groups, body, 0)

    return k(fused, idx3)


def kernel(color_grid, valid_mask, coord_color_table, row_table, col_table,
           W_color, b_color, W_valid, b_valid, ln_gamma, ln_beta):
    bsz, h, w = color_grid.shape
    f32 = jnp.float32

    bias = (b_color + b_valid).astype(f32)
    wc_rows = W_color.T.astype(f32)
    wv_row = W_valid[:, 0].astype(f32)
    cc = (col_table[:, None, :] + wc_rows[None, :, :]).reshape(_ROWS_PER_R, _HIDDEN)
    cc = cc + (wv_row + bias)[None, :]
    colb = col_table + bias[None, :]
    gamma2 = ln_gamma.reshape(1, _HIDDEN).astype(f32)
    beta2 = ln_beta.reshape(1, _HIDDEN).astype(f32)
    tbl = coord_color_table[:_NVALID].astype(f32)

    fused_v, fused_i = _build_fused_tables(tbl, row_table.astype(f32), cc, colb,
                                           gamma2, beta2)
    fused = jnp.concatenate([fused_v, fused_i], axis=0)

    r_ids = jnp.arange(h, dtype=jnp.int32)
    c_ids = jnp.arange(w, dtype=jnp.int32)
    cg_t = color_grid.astype(jnp.int32).transpose(1, 2, 0)
    vm_t = valid_mask.transpose(1, 2, 0)
    token = (r_ids[:, None, None] * (_MAX_COLS * _NUM_COLORS)
             + c_ids[None, :, None] * _NUM_COLORS
             + cg_t)
    inv = _NVALID + r_ids[:, None, None] * _MAX_COLS + c_ids[None, :, None]
    idx_t = jnp.where(vm_t, token, inv).astype(jnp.int32).reshape(-1)

    rows = _sc_gather(fused, idx_t)
    return rows.reshape(h * w, bsz, _HIDDEN).swapaxes(0, 1)
